# Initial kernel scaffold; baseline (speedup 1.0000x reference)
import jax
import jax.numpy as jnp
from jax.experimental import pallas as pl


def kernel(codes, centroids):
    raise NotImplementedError("write your pallas kernel here")


# speedup vs baseline: 1.4897x; 1.4897x over previous
import jax
import jax.numpy as jnp
from jax.experimental import pallas as pl
from jax.experimental.pallas import tpu as pltpu

_BM = 1024


def _body(n_classes, codes_ref, cents_ref, out_ref, cents_nb):
    @pl.when(pl.program_id(0) == 0)
    def _():
        cents = cents_ref[...]
        nrm = jnp.sqrt(jnp.sum(cents * cents, axis=0, keepdims=True))
        cents_nb[...] = (cents / jnp.maximum(nrm, 1e-12)).astype(jnp.bfloat16)

    codes = codes_ref[...]
    nrm = jnp.sqrt(jnp.sum(codes * codes, axis=1, keepdims=True))
    codes_n = (codes / jnp.maximum(nrm, 1e-12)).astype(jnp.bfloat16)
    sim = jax.lax.dot_general(
        codes_n, cents_nb[...],
        dimension_numbers=(((1,), (0,)), ((), ())),
        preferred_element_type=jnp.float32)
    c = n_classes
    m = jnp.maximum(jnp.maximum(sim[:, 0 * c:1 * c], sim[:, 1 * c:2 * c]),
                    jnp.maximum(sim[:, 2 * c:3 * c], sim[:, 3 * c:4 * c]))
    out_ref[...] = 1.0 - m


def kernel(codes, centroids):
    b, d = codes.shape
    c, k, _ = centroids.shape
    cents_t = centroids.transpose(2, 1, 0).reshape(d, k * c)
    import functools
    body = functools.partial(_body, c)
    return pl.pallas_call(
        body,
        grid=(b // _BM,),
        in_specs=[
            pl.BlockSpec((_BM, d), lambda i: (i, 0)),
            pl.BlockSpec((d, k * c), lambda i: (0, 0)),
        ],
        out_specs=pl.BlockSpec((_BM, c), lambda i: (i, 0)),
        out_shape=jax.ShapeDtypeStruct((b, c), jnp.float32),
        scratch_shapes=[pltpu.VMEM((d, k * c), jnp.bfloat16)],
    )(codes, cents_t)


# speedup vs baseline: 1.5929x; 1.0692x over previous
import jax
import jax.numpy as jnp
from jax.experimental import pallas as pl
from jax.experimental.pallas import tpu as pltpu

_BM = 1024


def _body(n_classes, codes_ref, cents_ref, out_ref, cents_nb):
    @pl.when(pl.program_id(0) == 0)
    def _():
        cents = cents_ref[...]
        inv = jax.lax.rsqrt(
            jnp.maximum(jnp.sum(cents * cents, axis=1, keepdims=True), 1e-24))
        cents_nb[...] = (cents * inv).astype(jnp.bfloat16)

    codes = codes_ref[...]
    inv = jax.lax.rsqrt(
        jnp.maximum(jnp.sum(codes * codes, axis=1, keepdims=True), 1e-24))
    codes_n = (codes * inv).astype(jnp.bfloat16)
    sim = jax.lax.dot_general(
        codes_n, cents_nb[...],
        dimension_numbers=(((1,), (1,)), ((), ())),
        preferred_element_type=jnp.float32)
    c = n_classes
    m = jnp.maximum(jnp.maximum(sim[:, 0 * c:1 * c], sim[:, 1 * c:2 * c]),
                    jnp.maximum(sim[:, 2 * c:3 * c], sim[:, 3 * c:4 * c]))
    out_ref[...] = 1.0 - m


def kernel(codes, centroids):
    b, d = codes.shape
    c, k, _ = centroids.shape
    cents_t = centroids.transpose(1, 0, 2).reshape(k * c, d)
    import functools
    body = functools.partial(_body, c)
    return pl.pallas_call(
        body,
        grid=(b // _BM,),
        in_specs=[
            pl.BlockSpec((_BM, d), lambda i: (i, 0)),
            pl.BlockSpec((k * c, d), lambda i: (0, 0)),
        ],
        out_specs=pl.BlockSpec((_BM, c), lambda i: (i, 0)),
        out_shape=jax.ShapeDtypeStruct((b, c), jnp.float32),
        scratch_shapes=[pltpu.VMEM((k * c, d), jnp.bfloat16)],
    )(codes, cents_t)


# speedup vs baseline: 1.6070x; 1.0089x over previous
import jax
import jax.numpy as jnp
from jax.experimental import pallas as pl
from jax.experimental.pallas import tpu as pltpu

_BM = 1024


def _body(n_classes, codes_ref, cents_ref, out_ref, cents_nb):
    @pl.when(pl.program_id(0) == 0)
    def _():
        cents = cents_ref[...]
        inv = jax.lax.rsqrt(
            jnp.maximum(jnp.sum(cents * cents, axis=1, keepdims=True), 1e-24))
        cents_nb[...] = (cents * inv).astype(jnp.bfloat16)

    codes = codes_ref[...]
    inv = jax.lax.rsqrt(
        jnp.maximum(jnp.sum(codes * codes, axis=1, keepdims=True), 1e-24))
    codes_n = (codes * inv).astype(jnp.bfloat16)
    c = n_classes
    dn = (((1,), (1,)), ((), ()))
    m = jax.lax.dot_general(codes_n, cents_nb[0 * c:1 * c, :], dn,
                            preferred_element_type=jnp.float32)
    for kk in range(1, 4):
        m = jnp.maximum(m, jax.lax.dot_general(
            codes_n, cents_nb[kk * c:(kk + 1) * c, :], dn,
            preferred_element_type=jnp.float32))
    out_ref[...] = 1.0 - m


def kernel(codes, centroids):
    b, d = codes.shape
    c, k, _ = centroids.shape
    cents_t = centroids.transpose(1, 0, 2).reshape(k * c, d)
    import functools
    body = functools.partial(_body, c)
    return pl.pallas_call(
        body,
        grid=(b // _BM,),
        in_specs=[
            pl.BlockSpec((_BM, d), lambda i: (i, 0)),
            pl.BlockSpec((k * c, d), lambda i: (0, 0)),
        ],
        out_specs=pl.BlockSpec((_BM, c), lambda i: (i, 0)),
        out_shape=jax.ShapeDtypeStruct((b, c), jnp.float32),
        scratch_shapes=[pltpu.VMEM((k * c, d), jnp.bfloat16)],
    )(codes, cents_t)


# speedup vs baseline: 1.8514x; 1.1521x over previous
import jax
import jax.numpy as jnp
from jax.experimental import pallas as pl
from jax.experimental.pallas import tpu as pltpu

_BM = 2048


def _body(n_classes, codes_ref, cents_ref, out_ref, cents_nb):
    @pl.when(pl.program_id(0) == 0)
    def _():
        cents = cents_ref[...]
        inv = jax.lax.rsqrt(
            jnp.maximum(jnp.sum(cents * cents, axis=1, keepdims=True), 1e-24))
        cents_nb[...] = (cents * inv).astype(jnp.bfloat16)

    codes = codes_ref[...]
    inv = jax.lax.rsqrt(
        jnp.maximum(jnp.sum(codes * codes, axis=1, keepdims=True), 1e-24))
    codes_n = (codes * inv).astype(jnp.bfloat16)
    c = n_classes
    dn = (((1,), (1,)), ((), ()))
    m = jax.lax.dot_general(codes_n, cents_nb[0 * c:1 * c, :], dn,
                            preferred_element_type=jnp.float32)
    for kk in range(1, 4):
        m = jnp.maximum(m, jax.lax.dot_general(
            codes_n, cents_nb[kk * c:(kk + 1) * c, :], dn,
            preferred_element_type=jnp.float32))
    out_ref[...] = 1.0 - m


def kernel(codes, centroids):
    b, d = codes.shape
    c, k, _ = centroids.shape
    cents_t = centroids.transpose(1, 0, 2).reshape(k * c, d)
    import functools
    body = functools.partial(_body, c)
    return pl.pallas_call(
        body,
        grid=(b // _BM,),
        in_specs=[
            pl.BlockSpec((_BM, d), lambda i: (i, 0)),
            pl.BlockSpec((k * c, d), lambda i: (0, 0)),
        ],
        out_specs=pl.BlockSpec((_BM, c), lambda i: (i, 0)),
        out_shape=jax.ShapeDtypeStruct((b, c), jnp.float32),
        scratch_shapes=[pltpu.VMEM((k * c, d), jnp.bfloat16)],
    )(codes, cents_t)


# speedup vs baseline: 1.8619x; 1.0057x over previous
import jax
import jax.numpy as jnp
from jax.experimental import pallas as pl
from jax.experimental.pallas import tpu as pltpu

_BM = 4096


def _body(n_classes, codes_ref, cents_ref, out_ref, cents_nb):
    @pl.when(pl.program_id(0) == 0)
    def _():
        cents = cents_ref[...]
        inv = jax.lax.rsqrt(
            jnp.maximum(jnp.sum(cents * cents, axis=1, keepdims=True), 1e-24))
        cents_nb[...] = (cents * inv).astype(jnp.bfloat16)

    codes = codes_ref[...]
    inv = jax.lax.rsqrt(
        jnp.maximum(jnp.sum(codes * codes, axis=1, keepdims=True), 1e-24))
    codes_n = (codes * inv).astype(jnp.bfloat16)
    c = n_classes
    dn = (((1,), (1,)), ((), ()))
    m = jax.lax.dot_general(codes_n, cents_nb[0 * c:1 * c, :], dn,
                            preferred_element_type=jnp.float32)
    for kk in range(1, 4):
        m = jnp.maximum(m, jax.lax.dot_general(
            codes_n, cents_nb[kk * c:(kk + 1) * c, :], dn,
            preferred_element_type=jnp.float32))
    out_ref[...] = 1.0 - m


def kernel(codes, centroids):
    b, d = codes.shape
    c, k, _ = centroids.shape
    cents_t = centroids.transpose(1, 0, 2).reshape(k * c, d)
    import functools
    body = functools.partial(_body, c)
    return pl.pallas_call(
        body,
        grid=(b // _BM,),
        in_specs=[
            pl.BlockSpec((_BM, d), lambda i: (i, 0)),
            pl.BlockSpec((k * c, d), lambda i: (0, 0)),
        ],
        out_specs=pl.BlockSpec((_BM, c), lambda i: (i, 0)),
        out_shape=jax.ShapeDtypeStruct((b, c), jnp.float32),
        scratch_shapes=[pltpu.VMEM((k * c, d), jnp.bfloat16)],
    )(codes, cents_t)
